# Initial kernel scaffold; baseline (speedup 1.0000x reference)
import jax
import jax.numpy as jnp
from jax.experimental import pallas as pl


def kernel(x, edge_index, edge_attr, We, Wc, bc, W1, b1, W2, b2):
    raise NotImplementedError("write your pallas kernel here")


# speedup vs baseline: 1.2862x; 1.2862x over previous
import functools

import jax
import jax.numpy as jnp
from jax import lax
from jax.experimental import pallas as pl
from jax.experimental.pallas import tpu as pltpu
from jax.experimental.pallas import tpu_sc as plsc

N_NODES = 10000
N_EDGES = 160000
D_FEAT = 256
D_EDGE = 16
D_CONV = 128
D_HIDDEN = 1024

DH = D_FEAT // 2
N_TILES = 16
CHUNK = 128
CHUNKS_PER_TILE = -(-N_EDGES // (N_TILES * CHUNK))
TILE_E = CHUNKS_PER_TILE * CHUNK
E_PAD = TILE_E * N_TILES
R_ACC = 10112
INIT_ROWS = R_ACC // N_TILES

BE = 512
BN = 1000


def _proj_body(e_ref, w_ref, p_ref):
    p_ref[0] = jnp.dot(e_ref[...], w_ref[0], preferred_element_type=jnp.float32)


def _edge_proj(ea_pad, We_s):
    return pl.pallas_call(
        _proj_body,
        grid=(2, E_PAD // BE),
        in_specs=[
            pl.BlockSpec((BE, D_EDGE), lambda h, i: (i, 0)),
            pl.BlockSpec((1, D_EDGE, DH), lambda h, i: (h, 0, 0)),
        ],
        out_specs=pl.BlockSpec((1, BE, DH), lambda h, i: (h, i, 0)),
        out_shape=jax.ShapeDtypeStruct((2, E_PAD, DH), jnp.float32),
    )(ea_pad, We_s)


def _sc_body(xflat, src, dst, pflat, out, acc, srcb, srcadj, dstb, gbuf, pbuf, sem):
    c = lax.axis_index("c")
    s = lax.axis_index("s")

    init0 = s * INIT_ROWS
    pltpu.sync_copy(xflat.at[pl.ds(c * R_ACC + init0, INIT_ROWS)],
                    acc.at[pl.ds(init0, INIT_ROWS)])
    plsc.subcore_barrier()

    tbase = s * TILE_E
    xoff = c * R_ACC
    poff = c * E_PAD

    def chunk_body(i, carry):
        base = tbase + i * CHUNK
        pltpu.sync_copy(src.at[pl.ds(base, CHUNK)], srcb)
        pltpu.sync_copy(dst.at[pl.ds(base, CHUNK)], dstb)
        pltpu.sync_copy(pflat.at[pl.ds(poff + base, CHUNK)], pbuf)
        for j in range(CHUNK // 16):
            srcadj[pl.ds(j * 16, 16)] = srcb[pl.ds(j * 16, 16)] + xoff
        pltpu.async_copy(xflat.at[srcadj], gbuf, sem).wait()

        def row_body(r, carry2):
            for j in range(DH // 16):
                v = gbuf[r, pl.ds(j * 16, 16)] + pbuf[r, pl.ds(j * 16, 16)]
                gbuf[r, pl.ds(j * 16, 16)] = jnp.maximum(v, 0.0)
            return carry2

        lax.fori_loop(0, CHUNK, row_body, 0)
        pltpu.sync_copy(gbuf, acc.at[dstb], add=True)
        return carry

    lax.fori_loop(0, CHUNKS_PER_TILE, chunk_body, 0)
    plsc.subcore_barrier()

    pltpu.sync_copy(acc.at[pl.ds(init0, INIT_ROWS)], out.at[c, pl.ds(init0, INIT_ROWS)])


_sc_edge = functools.partial(
    pl.kernel,
    out_type=jax.ShapeDtypeStruct((2, R_ACC, DH), jnp.float32),
    mesh=plsc.VectorSubcoreMesh(core_axis_name="c", subcore_axis_name="s"),
    scratch_types=[
        pltpu.VMEM_SHARED((R_ACC, DH), jnp.float32),
        pltpu.VMEM((CHUNK,), jnp.int32),
        pltpu.VMEM((CHUNK,), jnp.int32),
        pltpu.VMEM((CHUNK,), jnp.int32),
        pltpu.VMEM((CHUNK, DH), jnp.float32),
        pltpu.VMEM((CHUNK, DH), jnp.float32),
        pltpu.SemaphoreType.DMA,
    ],
)(_sc_body)


def _mlp_body(a_ref, wc_ref, bc_ref, w1_ref, b1_ref, w2_ref, b2_ref, o_ref):
    h = jnp.dot(a_ref[0], wc_ref[0], preferred_element_type=jnp.float32)
    h += jnp.dot(a_ref[1], wc_ref[1], preferred_element_type=jnp.float32)
    h += bc_ref[0]
    h1 = jnp.maximum(jnp.dot(h, w1_ref[...], preferred_element_type=jnp.float32) + b1_ref[0], 0.0)
    o_ref[...] = jnp.dot(h1, w2_ref[...], preferred_element_type=jnp.float32) + b2_ref[0]


def _mlp(agg, Wc_s, bc, W1, b1, W2, b2):
    return pl.pallas_call(
        _mlp_body,
        grid=(N_NODES // BN,),
        in_specs=[
            pl.BlockSpec((2, BN, DH), lambda i: (0, i, 0)),
            pl.BlockSpec((2, DH, D_CONV), lambda i: (0, 0, 0)),
            pl.BlockSpec((1, D_CONV), lambda i: (0, 0)),
            pl.BlockSpec((D_CONV, D_HIDDEN), lambda i: (0, 0)),
            pl.BlockSpec((1, D_HIDDEN), lambda i: (0, 0)),
            pl.BlockSpec((D_HIDDEN, D_FEAT), lambda i: (0, 0)),
            pl.BlockSpec((1, D_FEAT), lambda i: (0, 0)),
        ],
        out_specs=pl.BlockSpec((BN, D_FEAT), lambda i: (i, 0)),
        out_shape=jax.ShapeDtypeStruct((N_NODES, D_FEAT), jnp.float32),
    )(agg, Wc_s, bc.reshape(1, -1), W1, b1.reshape(1, -1), W2, b2.reshape(1, -1))


def kernel(x, edge_index, edge_attr, We, Wc, bc, W1, b1, W2, b2):
    src = edge_index[0]
    dst = edge_index[1]
    pad_e = E_PAD - N_EDGES

    x0 = x[:, :DH]
    x1 = x[:, DH:]
    zrows = jnp.zeros((R_ACC - N_NODES, DH), jnp.float32)
    xflat = jnp.concatenate([x0, zrows, x1, zrows], axis=0)

    src_pad = jnp.concatenate([src, jnp.zeros((pad_e,), jnp.int32)])
    dst_pad = jnp.concatenate([dst, jnp.full((pad_e,), N_NODES, jnp.int32)])
    ea_pad = jnp.concatenate([edge_attr, jnp.zeros((pad_e, D_EDGE), jnp.float32)], axis=0)

    We_s = jnp.stack([We[:, :DH], We[:, DH:]])
    Wc_s = jnp.stack([Wc[:DH], Wc[DH:]])

    P = _edge_proj(ea_pad, We_s)
    pflat = P.reshape(2 * E_PAD, DH)
    agg = _sc_edge(xflat, src_pad, dst_pad, pflat)
    return _mlp(agg, Wc_s, bc, W1, b1, W2, b2)


# speedup vs baseline: 1.5759x; 1.2253x over previous
import functools

import jax
import jax.numpy as jnp
from jax import lax
from jax.experimental import pallas as pl
from jax.experimental.pallas import tpu as pltpu
from jax.experimental.pallas import tpu_sc as plsc

N_NODES = 10000
N_EDGES = 160000
D_FEAT = 256
D_EDGE = 16
D_CONV = 128
D_HIDDEN = 1024

DH = D_FEAT // 2
N_TILES = 16
CHUNK = 96
CHUNKS_PER_TILE = 105
TILE_E = CHUNKS_PER_TILE * CHUNK
E_PAD = TILE_E * N_TILES
R_ACC = 10112
INIT_ROWS = R_ACC // N_TILES

BE = 512
BN = 1000


def _proj_body(e_ref, w_ref, p_ref):
    p_ref[0] = jnp.dot(e_ref[...], w_ref[0], preferred_element_type=jnp.float32)


def _edge_proj(ea_pad, We_s):
    return pl.pallas_call(
        _proj_body,
        grid=(2, E_PAD // BE),
        in_specs=[
            pl.BlockSpec((BE, D_EDGE), lambda h, i: (i, 0)),
            pl.BlockSpec((1, D_EDGE, DH), lambda h, i: (h, 0, 0)),
        ],
        out_specs=pl.BlockSpec((1, BE, DH), lambda h, i: (h, i, 0)),
        out_shape=jax.ShapeDtypeStruct((2, E_PAD, DH), jnp.float32),
    )(ea_pad, We_s)


def _sc_body(xflat, src, dst, pflat, out, acc,
             srcb, dstb, gbuf, pbuf, sio, sg):
    c = lax.axis_index("c")
    s = lax.axis_index("s")

    init0 = s * INIT_ROWS
    pltpu.sync_copy(xflat.at[pl.ds(c * R_ACC + init0, INIT_ROWS)],
                    acc.at[pl.ds(init0, INIT_ROWS)])
    plsc.subcore_barrier()

    tbase = s * TILE_E
    xoff = c * R_ACC
    poff = c * E_PAD

    def issue_idx(i, b2):
        base = tbase + i * CHUNK
        pltpu.async_copy(src.at[pl.ds(base, CHUNK)], srcb[b2], sio[b2])
        pltpu.async_copy(dst.at[pl.ds(base, CHUNK)], dstb[b2], sio[b2])
        pltpu.async_copy(pflat.at[pl.ds(poff + base, CHUNK)], pbuf[b2], sio[b2])

    def wait_idx(b2):
        pltpu.make_async_copy(src.at[pl.ds(0, CHUNK)], srcb[b2], sio[b2]).wait()
        pltpu.make_async_copy(dst.at[pl.ds(0, CHUNK)], dstb[b2], sio[b2]).wait()
        pltpu.make_async_copy(pflat.at[pl.ds(0, CHUNK)], pbuf[b2], sio[b2]).wait()

    def issue_gather(b2):
        for j in range(CHUNK // 16):
            srcb[b2][pl.ds(j * 16, 16)] = srcb[b2][pl.ds(j * 16, 16)] + xoff
        pltpu.async_copy(xflat.at[srcb[b2]], gbuf[b2], sg[b2])

    def wait_gather(b2):
        pltpu.make_async_copy(xflat.at[srcb[b2]], gbuf[b2], sg[b2]).wait()

    def compute(b2):
        def row_body(r):
            for j in range(DH // 16):
                v = gbuf[b2][r, pl.ds(j * 16, 16)] + pbuf[b2][r, pl.ds(j * 16, 16)]
                gbuf[b2][r, pl.ds(j * 16, 16)] = jnp.maximum(v, 0.0)

        plsc.parallel_loop(0, CHUNK, unroll=4)(row_body)

    def scatter(b2):
        pltpu.sync_copy(gbuf[b2], acc.at[dstb[b2]], add=True)

    def step(i, b2, do_idx=True, do_next_gather=True):
        wait_gather(b2)
        compute(b2)
        scatter(b2)
        if do_idx:
            issue_idx(i + 2, b2)
        if do_next_gather:
            wait_idx((b2 + 1) % 2)
            issue_gather((b2 + 1) % 2)

    issue_idx(0, 0)
    issue_idx(1, 1)
    wait_idx(0)
    issue_gather(0)

    def group_body(g, carry):
        i0 = g * 2
        step(i0, 0)
        step(i0 + 1, 1)
        return carry

    n_groups = (CHUNKS_PER_TILE - 3) // 2
    lax.fori_loop(0, n_groups, group_body, 0)

    i0 = n_groups * 2
    for i in range(i0, CHUNKS_PER_TILE):
        step(i, i % 2,
             do_idx=(i + 2 < CHUNKS_PER_TILE),
             do_next_gather=(i + 1 < CHUNKS_PER_TILE))
    plsc.subcore_barrier()

    pltpu.sync_copy(acc.at[pl.ds(init0, INIT_ROWS)], out.at[c, pl.ds(init0, INIT_ROWS)])


_sc_edge = functools.partial(
    pl.kernel,
    out_type=jax.ShapeDtypeStruct((2, R_ACC, DH), jnp.float32),
    mesh=plsc.VectorSubcoreMesh(core_axis_name="c", subcore_axis_name="s"),
    scratch_types=[
        pltpu.VMEM_SHARED((R_ACC, DH), jnp.float32),
        [pltpu.VMEM((CHUNK,), jnp.int32) for _ in range(2)],
        [pltpu.VMEM((CHUNK,), jnp.int32) for _ in range(2)],
        [pltpu.VMEM((CHUNK, DH), jnp.float32) for _ in range(2)],
        [pltpu.VMEM((CHUNK, DH), jnp.float32) for _ in range(2)],
        [pltpu.SemaphoreType.DMA for _ in range(2)],
        [pltpu.SemaphoreType.DMA for _ in range(2)],
    ],
)(_sc_body)


def _mlp_body(a_ref, wc_ref, bc_ref, w1_ref, b1_ref, w2_ref, b2_ref, o_ref):
    h = jnp.dot(a_ref[0], wc_ref[0], preferred_element_type=jnp.float32)
    h += jnp.dot(a_ref[1], wc_ref[1], preferred_element_type=jnp.float32)
    h += bc_ref[0]
    h1 = jnp.maximum(jnp.dot(h, w1_ref[...], preferred_element_type=jnp.float32) + b1_ref[0], 0.0)
    o_ref[...] = jnp.dot(h1, w2_ref[...], preferred_element_type=jnp.float32) + b2_ref[0]


def _mlp(agg, Wc_s, bc, W1, b1, W2, b2):
    return pl.pallas_call(
        _mlp_body,
        grid=(N_NODES // BN,),
        in_specs=[
            pl.BlockSpec((2, BN, DH), lambda i: (0, i, 0)),
            pl.BlockSpec((2, DH, D_CONV), lambda i: (0, 0, 0)),
            pl.BlockSpec((1, D_CONV), lambda i: (0, 0)),
            pl.BlockSpec((D_CONV, D_HIDDEN), lambda i: (0, 0)),
            pl.BlockSpec((1, D_HIDDEN), lambda i: (0, 0)),
            pl.BlockSpec((D_HIDDEN, D_FEAT), lambda i: (0, 0)),
            pl.BlockSpec((1, D_FEAT), lambda i: (0, 0)),
        ],
        out_specs=pl.BlockSpec((BN, D_FEAT), lambda i: (i, 0)),
        out_shape=jax.ShapeDtypeStruct((N_NODES, D_FEAT), jnp.float32),
    )(agg, Wc_s, bc.reshape(1, -1), W1, b1.reshape(1, -1), W2, b2.reshape(1, -1))


def kernel(x, edge_index, edge_attr, We, Wc, bc, W1, b1, W2, b2):
    src = edge_index[0]
    dst = edge_index[1]
    pad_e = E_PAD - N_EDGES

    x0 = x[:, :DH]
    x1 = x[:, DH:]
    zrows = jnp.zeros((R_ACC - N_NODES, DH), jnp.float32)
    xflat = jnp.concatenate([x0, zrows, x1, zrows], axis=0)

    pad_ids = jnp.arange(pad_e, dtype=jnp.int32)
    src_pad = jnp.concatenate([src, pad_ids % N_NODES])
    dst_pad = jnp.concatenate([dst, N_NODES + pad_ids % (R_ACC - N_NODES)])
    ea_pad = jnp.concatenate([edge_attr, jnp.zeros((pad_e, D_EDGE), jnp.float32)], axis=0)

    We_s = jnp.stack([We[:, :DH], We[:, DH:]])
    Wc_s = jnp.stack([Wc[:DH], Wc[DH:]])

    P = _edge_proj(ea_pad, We_s)
    pflat = P.reshape(2 * E_PAD, DH)
    agg = _sc_edge(xflat, src_pad, dst_pad, pflat)
    return _mlp(agg, Wc_s, bc, W1, b1, W2, b2)


# speedup vs baseline: 1.6643x; 1.0561x over previous
import functools

import jax
import jax.numpy as jnp
from jax import lax
from jax.experimental import pallas as pl
from jax.experimental.pallas import tpu as pltpu
from jax.experimental.pallas import tpu_sc as plsc

N_NODES = 10000
N_EDGES = 160000
D_FEAT = 256
D_EDGE = 16
D_CONV = 128
D_HIDDEN = 1024

DH = D_FEAT // 2
N_TILES = 16
CHUNK = 96
CHUNKS_PER_TILE = 105
TILE_E = CHUNKS_PER_TILE * CHUNK
E_PAD = TILE_E * N_TILES
R_ACC = 10112
INIT_ROWS = R_ACC // N_TILES

BE = 512
BN = 1000


def _proj_body(e_ref, w_ref, p_ref):
    p_ref[0] = jnp.dot(e_ref[...], w_ref[0], preferred_element_type=jnp.float32)


def _edge_proj(ea_pad, We_s):
    return pl.pallas_call(
        _proj_body,
        grid=(2, E_PAD // BE),
        in_specs=[
            pl.BlockSpec((BE, D_EDGE), lambda h, i: (i, 0)),
            pl.BlockSpec((1, D_EDGE, DH), lambda h, i: (h, 0, 0)),
        ],
        out_specs=pl.BlockSpec((1, BE, DH), lambda h, i: (h, i, 0)),
        out_shape=jax.ShapeDtypeStruct((2, E_PAD, DH), jnp.float32),
    )(ea_pad, We_s)


def _sc_body(xflat, src, dst, pflat, out, acc,
             srcb, dstb, gbuf, pbuf, sio, sg, ss):
    c = lax.axis_index("c")
    s = lax.axis_index("s")

    init0 = s * INIT_ROWS
    pltpu.sync_copy(xflat.at[pl.ds(c * R_ACC + init0, INIT_ROWS)],
                    acc.at[pl.ds(init0, INIT_ROWS)])
    plsc.subcore_barrier()

    tbase = s * TILE_E
    xoff = c * R_ACC
    poff = c * E_PAD

    def issue_idx(i, b2, b3):
        base = tbase + i * CHUNK
        pltpu.async_copy(src.at[pl.ds(base, CHUNK)], srcb[b2], sio[b2])
        pltpu.async_copy(dst.at[pl.ds(base, CHUNK)], dstb[b3], sio[b2])
        pltpu.async_copy(pflat.at[pl.ds(poff + base, CHUNK)], pbuf[b2], sio[b2])

    def wait_idx(b2, b3):
        pltpu.make_async_copy(src.at[pl.ds(0, CHUNK)], srcb[b2], sio[b2]).wait()
        pltpu.make_async_copy(dst.at[pl.ds(0, CHUNK)], dstb[b3], sio[b2]).wait()
        pltpu.make_async_copy(pflat.at[pl.ds(0, CHUNK)], pbuf[b2], sio[b2]).wait()

    def issue_gather(b2):
        for j in range(CHUNK // 16):
            srcb[b2][pl.ds(j * 16, 16)] = srcb[b2][pl.ds(j * 16, 16)] + xoff
        pltpu.async_copy(xflat.at[srcb[b2]], gbuf[b2], sg[b2])

    def wait_gather(b2):
        pltpu.make_async_copy(xflat.at[srcb[b2]], gbuf[b2], sg[b2]).wait()

    def compute(b2):
        def row_body(r):
            for j in range(DH // 16):
                v = gbuf[b2][r, pl.ds(j * 16, 16)] + pbuf[b2][r, pl.ds(j * 16, 16)]
                gbuf[b2][r, pl.ds(j * 16, 16)] = jnp.maximum(v, 0.0)

        plsc.parallel_loop(0, CHUNK, unroll=4)(row_body)

    def issue_scatter(b2, b3):
        pltpu.async_copy(gbuf[b2], acc.at[dstb[b3]], ss, add=True)

    def wait_scatter(b2, b3):
        pltpu.make_async_copy(gbuf[b2], acc.at[dstb[b3]], ss).wait()

    def step(i, b2, b3, first=False, do_idx=True, do_next_gather=True):
        wait_gather(b2)
        compute(b2)
        if not first:
            wait_scatter((b2 + 1) % 2, (b3 + 2) % 3)
        issue_scatter(b2, b3)
        if do_idx:
            issue_idx(i + 2, b2, (b3 + 2) % 3)
        if do_next_gather:
            wait_idx((b2 + 1) % 2, (b3 + 1) % 3)
            issue_gather((b2 + 1) % 2)

    issue_idx(0, 0, 0)
    issue_idx(1, 1, 1)
    wait_idx(0, 0)
    issue_gather(0)
    step(0, 0, 0, first=True)
    step(1, 1, 1)

    def group_body(g, carry):
        i0 = 2 + g * 6
        for k in range(6):
            step(i0 + k, k % 2, (2 + k) % 3)
        return carry

    n_groups = (CHUNKS_PER_TILE - 9) // 6
    lax.fori_loop(0, n_groups, group_body, 0)

    i0 = 2 + n_groups * 6
    for i in range(i0, CHUNKS_PER_TILE):
        step(i, i % 2, i % 3,
             do_idx=(i + 2 < CHUNKS_PER_TILE),
             do_next_gather=(i + 1 < CHUNKS_PER_TILE))
    wait_scatter((CHUNKS_PER_TILE - 1) % 2, (CHUNKS_PER_TILE - 1) % 3)
    plsc.subcore_barrier()

    pltpu.sync_copy(acc.at[pl.ds(init0, INIT_ROWS)], out.at[c, pl.ds(init0, INIT_ROWS)])


_sc_edge = functools.partial(
    pl.kernel,
    out_type=jax.ShapeDtypeStruct((2, R_ACC, DH), jnp.float32),
    mesh=plsc.VectorSubcoreMesh(core_axis_name="c", subcore_axis_name="s"),
    scratch_types=[
        pltpu.VMEM_SHARED((R_ACC, DH), jnp.float32),
        [pltpu.VMEM((CHUNK,), jnp.int32) for _ in range(2)],
        [pltpu.VMEM((CHUNK,), jnp.int32) for _ in range(3)],
        [pltpu.VMEM((CHUNK, DH), jnp.float32) for _ in range(2)],
        [pltpu.VMEM((CHUNK, DH), jnp.float32) for _ in range(2)],
        [pltpu.SemaphoreType.DMA for _ in range(2)],
        [pltpu.SemaphoreType.DMA for _ in range(2)],
        pltpu.SemaphoreType.DMA,
    ],
)(_sc_body)


def _mlp_body(a_ref, wc_ref, bc_ref, w1_ref, b1_ref, w2_ref, b2_ref, o_ref):
    h = jnp.dot(a_ref[0], wc_ref[0], preferred_element_type=jnp.float32)
    h += jnp.dot(a_ref[1], wc_ref[1], preferred_element_type=jnp.float32)
    h += bc_ref[0]
    h1 = jnp.maximum(jnp.dot(h, w1_ref[...], preferred_element_type=jnp.float32) + b1_ref[0], 0.0)
    o_ref[...] = jnp.dot(h1, w2_ref[...], preferred_element_type=jnp.float32) + b2_ref[0]


def _mlp(agg, Wc_s, bc, W1, b1, W2, b2):
    return pl.pallas_call(
        _mlp_body,
        grid=(N_NODES // BN,),
        in_specs=[
            pl.BlockSpec((2, BN, DH), lambda i: (0, i, 0)),
            pl.BlockSpec((2, DH, D_CONV), lambda i: (0, 0, 0)),
            pl.BlockSpec((1, D_CONV), lambda i: (0, 0)),
            pl.BlockSpec((D_CONV, D_HIDDEN), lambda i: (0, 0)),
            pl.BlockSpec((1, D_HIDDEN), lambda i: (0, 0)),
            pl.BlockSpec((D_HIDDEN, D_FEAT), lambda i: (0, 0)),
            pl.BlockSpec((1, D_FEAT), lambda i: (0, 0)),
        ],
        out_specs=pl.BlockSpec((BN, D_FEAT), lambda i: (i, 0)),
        out_shape=jax.ShapeDtypeStruct((N_NODES, D_FEAT), jnp.float32),
    )(agg, Wc_s, bc.reshape(1, -1), W1, b1.reshape(1, -1), W2, b2.reshape(1, -1))


def kernel(x, edge_index, edge_attr, We, Wc, bc, W1, b1, W2, b2):
    src = edge_index[0]
    dst = edge_index[1]
    pad_e = E_PAD - N_EDGES

    x0 = x[:, :DH]
    x1 = x[:, DH:]
    zrows = jnp.zeros((R_ACC - N_NODES, DH), jnp.float32)
    xflat = jnp.concatenate([x0, zrows, x1, zrows], axis=0)

    pad_ids = jnp.arange(pad_e, dtype=jnp.int32)
    src_pad = jnp.concatenate([src, pad_ids % N_NODES])
    dst_pad = jnp.concatenate([dst, N_NODES + pad_ids % (R_ACC - N_NODES)])
    ea_pad = jnp.concatenate([edge_attr, jnp.zeros((pad_e, D_EDGE), jnp.float32)], axis=0)

    We_s = jnp.stack([We[:, :DH], We[:, DH:]])
    Wc_s = jnp.stack([Wc[:DH], Wc[DH:]])

    P = _edge_proj(ea_pad, We_s)
    pflat = P.reshape(2 * E_PAD, DH)
    agg = _sc_edge(xflat, src_pad, dst_pad, pflat)
    return _mlp(agg, Wc_s, bc, W1, b1, W2, b2)
